# Initial kernel scaffold; baseline (speedup 1.0000x reference)
import jax
import jax.numpy as jnp
from jax.experimental import pallas as pl


def kernel(x, edge_index, W_mu, b_mu, W_logstd, b_logstd):
    raise NotImplementedError("write your pallas kernel here")


# speedup vs baseline: 36.5147x; 36.5147x over previous
import functools

import jax
import jax.numpy as jnp
from jax import lax
from jax.experimental import pallas as pl
from jax.experimental.pallas import tpu as pltpu
from jax.experimental.pallas import tpu_sc as plsc

N = 10000
E = 320000
IN_CH = 128
CH2 = 32
R = 10240
NW = 32
CHUNK = 128
KCH = 79
E_PAD = NW * KCH * CHUNK
ROWS_PER_TILE = R // 16
NBLK = 16
DEGW = 16


@functools.cache
def _sc_edge_agg(width):
    mesh = plsc.VectorSubcoreMesh(core_axis_name="c", subcore_axis_name="s")

    @functools.partial(
        pl.kernel,
        mesh=mesh,
        compiler_params=pltpu.CompilerParams(use_tc_tiling_on_sc=False),
        out_type=jax.ShapeDtypeStruct((2, R, width), jnp.float32),
        scratch_types=[
            pltpu.VMEM((KCH, CHUNK), jnp.int32),
            pltpu.VMEM((KCH, CHUNK), jnp.int32),
            pltpu.VMEM((CHUNK, width), jnp.float32),
            pltpu.VMEM_SHARED((R, width), jnp.float32),
            pltpu.SemaphoreType.DMA,
        ],
    )
    def k(table, srci, dsti, zer, out, srcv, dstv, rows, acc, sem):
        c = lax.axis_index("c")
        s = lax.axis_index("s")
        wid = s * 2 + c
        sl = pl.ds(s * ROWS_PER_TILE, ROWS_PER_TILE)
        pltpu.sync_copy(zer.at[sl], acc.at[sl])
        pltpu.sync_copy(srci.at[wid], srcv)
        pltpu.sync_copy(dsti.at[wid], dstv)
        plsc.subcore_barrier()

        def body(j, carry):
            pltpu.async_copy(table.at[srcv.at[j]], rows, sem).wait()
            pltpu.sync_copy(rows, acc.at[dstv.at[j]], add=True)
            return carry

        lax.fori_loop(0, KCH, body, 0)
        plsc.subcore_barrier()
        pltpu.sync_copy(acc.at[sl], out.at[c, sl])

    return k


def _tc_matmul_hs(x_pad, w_all, degp):
    def body(x_ref, w_ref, d_ref, hs_ref, dis_ref):
        deg = d_ref[0, :, :1] + d_ref[1, :, :1] + 1.0
        dis = lax.rsqrt(deg)
        h = jnp.dot(x_ref[...], w_ref[...], preferred_element_type=jnp.float32)
        hs_ref[...] = h * dis
        dis_ref[...] = dis

    return pl.pallas_call(
        body,
        grid=(NBLK,),
        in_specs=[
            pl.BlockSpec((ROWS_PER_TILE, IN_CH), lambda i: (i, 0)),
            pl.BlockSpec((IN_CH, CH2), lambda i: (0, 0)),
            pl.BlockSpec((2, ROWS_PER_TILE, DEGW), lambda i: (0, i, 0)),
        ],
        out_specs=[
            pl.BlockSpec((ROWS_PER_TILE, CH2), lambda i: (i, 0)),
            pl.BlockSpec((ROWS_PER_TILE, 1), lambda i: (i, 0)),
        ],
        out_shape=[
            jax.ShapeDtypeStruct((R, CH2), jnp.float32),
            jax.ShapeDtypeStruct((R, 1), jnp.float32),
        ],
    )(x_pad, w_all, degp)


def _tc_finalize(agg, hs, dis, b_all):
    def body(a_ref, hs_ref, dis_ref, b_ref, o_ref):
        o_ref[...] = dis_ref[...] * (a_ref[0] + a_ref[1] + hs_ref[...]) + b_ref[...]

    return pl.pallas_call(
        body,
        grid=(NBLK,),
        in_specs=[
            pl.BlockSpec((2, ROWS_PER_TILE, CH2), lambda i: (0, i, 0)),
            pl.BlockSpec((ROWS_PER_TILE, CH2), lambda i: (i, 0)),
            pl.BlockSpec((ROWS_PER_TILE, 1), lambda i: (i, 0)),
            pl.BlockSpec((1, CH2), lambda i: (0, 0)),
        ],
        out_specs=pl.BlockSpec((ROWS_PER_TILE, CH2), lambda i: (i, 0)),
        out_shape=jax.ShapeDtypeStruct((R, CH2), jnp.float32),
    )(agg, hs, dis, b_all)


def kernel(x, edge_index, W_mu, b_mu, W_logstd, b_logstd):
    src = edge_index[0]
    dst = edge_index[1]
    npad = E_PAD - E
    pad_ids = (N + jnp.arange(npad, dtype=jnp.int32) % (R - N)).astype(jnp.int32)
    srcp = jnp.concatenate([src, pad_ids]).reshape(NW, KCH, CHUNK)
    dstp = jnp.concatenate([dst, pad_ids]).reshape(NW, KCH, CHUNK)

    ones_t = jnp.ones((R, DEGW), jnp.float32)
    zer1 = jnp.zeros((R, DEGW), jnp.float32)
    degp = _sc_edge_agg(DEGW)(ones_t, srcp, dstp, zer1)

    w_all = jnp.concatenate([W_mu, W_logstd], axis=0).T
    x_pad = jnp.pad(x, ((0, R - N), (0, 0)))
    hs, dis = _tc_matmul_hs(x_pad, w_all, degp)

    zer32 = jnp.zeros((R, CH2), jnp.float32)
    agg = _sc_edge_agg(CH2)(hs, srcp, dstp, zer32)

    b_all = jnp.concatenate([b_mu, b_logstd]).reshape(1, CH2)
    outp = _tc_finalize(agg, hs, dis, b_all)
    return (outp[:N, :16], outp[:N, 16:])


# speedup vs baseline: 62.1438x; 1.7019x over previous
import functools

import jax
import jax.numpy as jnp
from jax import lax
from jax.experimental import pallas as pl
from jax.experimental.pallas import tpu as pltpu
from jax.experimental.pallas import tpu_sc as plsc

N = 10000
E = 320000
IN_CH = 128
CH2 = 32
R = 10240
NW = 32
CHUNK = 128
KCH = 80
KIDX = KCH + 2
E_PAD = NW * KIDX * CHUNK
ROWS_PER_TILE = R // 16
NBLK = 16

_SC_PARAMS = pltpu.CompilerParams(
    use_tc_tiling_on_sc=False, needs_layout_passes=False)


@functools.cache
def _sc_deg():
    mesh = plsc.VectorSubcoreMesh(core_axis_name="c", subcore_axis_name="s")

    @functools.partial(
        pl.kernel,
        mesh=mesh,
        compiler_params=_SC_PARAMS,
        out_type=jax.ShapeDtypeStruct((NW, R), jnp.float32),
        scratch_types=[
            pltpu.VMEM((KIDX, CHUNK), jnp.int32),
            pltpu.VMEM((R,), jnp.float32),
        ],
    )
    def k(dsti, zer, out, dstv, acc):
        c = lax.axis_index("c")
        s = lax.axis_index("s")
        wid = s * 2 + c
        pltpu.sync_copy(zer, acc)
        pltpu.sync_copy(dsti.at[wid], dstv)
        ones16 = jnp.ones((16,), jnp.float32)

        def body(j, carry):
            def inner(i, carry2):
                idx16 = dstv[j, pl.ds(i * 16, 16)]
                plsc.addupdate_scatter(acc, [idx16], ones16)
                return carry2

            return lax.fori_loop(0, CHUNK // 16, inner, carry)

        lax.fori_loop(0, KCH, body, 0)
        pltpu.sync_copy(acc, out.at[wid])

    return k


@functools.cache
def _sc_edge_agg():
    mesh = plsc.VectorSubcoreMesh(core_axis_name="c", subcore_axis_name="s")

    @functools.partial(
        pl.kernel,
        mesh=mesh,
        compiler_params=_SC_PARAMS,
        out_type=jax.ShapeDtypeStruct((2, R, CH2), jnp.float32),
        scratch_types=[
            pltpu.VMEM((KIDX, CHUNK), jnp.int32),
            pltpu.VMEM((KIDX, CHUNK), jnp.int32),
            pltpu.VMEM((CHUNK, CH2), jnp.float32),
            pltpu.VMEM((CHUNK, CH2), jnp.float32),
            pltpu.VMEM_SHARED((R, CH2), jnp.float32),
            pltpu.SemaphoreType.DMA,
            pltpu.SemaphoreType.DMA,
        ],
    )
    def k(table, srci, dsti, zer, out, srcv, dstv, rows0, rows1, acc, sem0, sem1):
        c = lax.axis_index("c")
        s = lax.axis_index("s")
        wid = s * 2 + c
        sl = pl.ds(s * ROWS_PER_TILE, ROWS_PER_TILE)
        pltpu.sync_copy(zer.at[sl], acc.at[sl])
        pltpu.sync_copy(srci.at[wid], srcv)
        pltpu.sync_copy(dsti.at[wid], dstv)
        plsc.subcore_barrier()

        pltpu.make_async_copy(table.at[srcv.at[0]], rows0, sem0).start()
        pltpu.make_async_copy(table.at[srcv.at[1]], rows1, sem1).start()

        def body(jj, carry):
            j = 2 * jj
            pltpu.make_async_copy(table.at[srcv.at[j]], rows0, sem0).wait()
            pltpu.sync_copy(rows0, acc.at[dstv.at[j]], add=True)
            pltpu.make_async_copy(table.at[srcv.at[j + 2]], rows0, sem0).start()
            pltpu.make_async_copy(table.at[srcv.at[j + 1]], rows1, sem1).wait()
            pltpu.sync_copy(rows1, acc.at[dstv.at[j + 1]], add=True)
            pltpu.make_async_copy(table.at[srcv.at[j + 3]], rows1, sem1).start()
            return carry

        lax.fori_loop(0, KCH // 2, body, 0)
        pltpu.make_async_copy(table.at[srcv.at[KCH]], rows0, sem0).wait()
        pltpu.make_async_copy(table.at[srcv.at[KCH + 1]], rows1, sem1).wait()
        plsc.subcore_barrier()
        pltpu.sync_copy(acc.at[sl], out.at[c, sl])

    return k


def _tc_matmul_hs(x_pad, w_all, degp):
    def body(x_ref, w_ref, d_ref, hs_ref, dis_ref):
        deg = jnp.sum(d_ref[...], axis=0)[:, None] + 1.0
        dis = lax.rsqrt(deg)
        h = jnp.dot(x_ref[...], w_ref[...], preferred_element_type=jnp.float32)
        hs_ref[...] = h * dis
        dis_ref[...] = dis

    return pl.pallas_call(
        body,
        grid=(NBLK,),
        in_specs=[
            pl.BlockSpec((ROWS_PER_TILE, IN_CH), lambda i: (i, 0)),
            pl.BlockSpec((IN_CH, CH2), lambda i: (0, 0)),
            pl.BlockSpec((NW, ROWS_PER_TILE), lambda i: (0, i)),
        ],
        out_specs=[
            pl.BlockSpec((ROWS_PER_TILE, CH2), lambda i: (i, 0)),
            pl.BlockSpec((ROWS_PER_TILE, 1), lambda i: (i, 0)),
        ],
        out_shape=[
            jax.ShapeDtypeStruct((R, CH2), jnp.float32),
            jax.ShapeDtypeStruct((R, 1), jnp.float32),
        ],
    )(x_pad, w_all, degp)


def _tc_finalize(agg, hs, dis, b_all):
    def body(a_ref, hs_ref, dis_ref, b_ref, o_ref):
        o_ref[...] = dis_ref[...] * (a_ref[0] + a_ref[1] + hs_ref[...]) + b_ref[...]

    return pl.pallas_call(
        body,
        grid=(NBLK,),
        in_specs=[
            pl.BlockSpec((2, ROWS_PER_TILE, CH2), lambda i: (0, i, 0)),
            pl.BlockSpec((ROWS_PER_TILE, CH2), lambda i: (i, 0)),
            pl.BlockSpec((ROWS_PER_TILE, 1), lambda i: (i, 0)),
            pl.BlockSpec((1, CH2), lambda i: (0, 0)),
        ],
        out_specs=pl.BlockSpec((ROWS_PER_TILE, CH2), lambda i: (i, 0)),
        out_shape=jax.ShapeDtypeStruct((R, CH2), jnp.float32),
    )(agg, hs, dis, b_all)


def kernel(x, edge_index, W_mu, b_mu, W_logstd, b_logstd):
    src = edge_index[0]
    dst = edge_index[1]
    npad = NW * KCH * CHUNK - E
    pad_ids = (N + jnp.arange(npad, dtype=jnp.int32) % (R - N)).astype(jnp.int32)
    dummy = (N + jnp.arange(NW * 2 * CHUNK, dtype=jnp.int32) % (R - N)).reshape(
        NW, 2, CHUNK)
    srcp = jnp.concatenate(
        [jnp.concatenate([src, pad_ids]).reshape(NW, KCH, CHUNK), dummy], axis=1)
    dstp = jnp.concatenate(
        [jnp.concatenate([dst, pad_ids]).reshape(NW, KCH, CHUNK), dummy], axis=1)

    zer1 = jnp.zeros((R,), jnp.float32)
    degp = _sc_deg()(dstp, zer1)

    w_all = jnp.concatenate([W_mu, W_logstd], axis=0).T
    x_pad = jnp.pad(x, ((0, R - N), (0, 0)))
    hs, dis = _tc_matmul_hs(x_pad, w_all, degp)

    zer32 = jnp.zeros((R, CH2), jnp.float32)
    agg = _sc_edge_agg()(hs, srcp, dstp, zer32)

    b_all = jnp.concatenate([b_mu, b_logstd]).reshape(1, CH2)
    outp = _tc_finalize(agg, hs, dis, b_all)
    return (outp[:N, :16], outp[:N, 16:])


# speedup vs baseline: 70.0933x; 1.1279x over previous
import functools

import jax
import jax.numpy as jnp
from jax import lax
from jax.experimental import pallas as pl
from jax.experimental.pallas import tpu as pltpu
from jax.experimental.pallas import tpu_sc as plsc

N = 10000
E = 320000
IN_CH = 128
CH2 = 32
R = 10240
NW = 32
CHUNK = 128
KCH = 80
KIDX = KCH + 2
E_PAD = NW * KIDX * CHUNK
ROWS_PER_TILE = R // 16
NBLK = 16

_SC_PARAMS = pltpu.CompilerParams(
    use_tc_tiling_on_sc=False, needs_layout_passes=False)


@functools.cache
def _sc_deg():
    mesh = plsc.VectorSubcoreMesh(core_axis_name="c", subcore_axis_name="s")

    @functools.partial(
        pl.kernel,
        mesh=mesh,
        compiler_params=_SC_PARAMS,
        out_type=jax.ShapeDtypeStruct((NW, R), jnp.float32),
        scratch_types=[
            pltpu.VMEM((KIDX, CHUNK), jnp.int32),
            pltpu.VMEM((R,), jnp.float32),
        ],
    )
    def k(edges, zer, out, dstv, acc):
        c = lax.axis_index("c")
        s = lax.axis_index("s")
        wid = s * 2 + c
        pltpu.sync_copy(zer, acc)
        pltpu.sync_copy(edges.at[1, wid], dstv)
        ones16 = jnp.ones((16,), jnp.float32)

        def body(j, carry):
            def inner(i, carry2):
                idx16 = dstv[j, pl.ds(i * 16, 16)]
                plsc.addupdate_scatter(acc, [idx16], ones16)
                return carry2

            return lax.fori_loop(0, CHUNK // 16, inner, carry)

        lax.fori_loop(0, KCH, body, 0)
        pltpu.sync_copy(acc, out.at[wid])

    return k


@functools.cache
def _sc_edge_agg():
    mesh = plsc.VectorSubcoreMesh(core_axis_name="c", subcore_axis_name="s")

    @functools.partial(
        pl.kernel,
        mesh=mesh,
        compiler_params=_SC_PARAMS,
        out_type=jax.ShapeDtypeStruct((2, R, CH2), jnp.float32),
        scratch_types=[
            pltpu.VMEM((KIDX, CHUNK), jnp.int32),
            pltpu.VMEM((KIDX, CHUNK), jnp.int32),
            pltpu.VMEM((CHUNK, CH2), jnp.float32),
            pltpu.VMEM((CHUNK, CH2), jnp.float32),
            pltpu.VMEM_SHARED((R, CH2), jnp.float32),
            pltpu.SemaphoreType.DMA,
            pltpu.SemaphoreType.DMA,
        ],
    )
    def k(table, edges, zer, out, srcv, dstv, rows0, rows1, acc, sem0, sem1):
        c = lax.axis_index("c")
        s = lax.axis_index("s")
        wid = s * 2 + c
        sl = pl.ds(s * ROWS_PER_TILE, ROWS_PER_TILE)
        pltpu.sync_copy(zer.at[sl], acc.at[sl])
        pltpu.sync_copy(edges.at[0, wid], srcv)
        pltpu.sync_copy(edges.at[1, wid], dstv)
        plsc.subcore_barrier()

        pltpu.make_async_copy(table.at[srcv.at[0]], rows0, sem0).start()
        pltpu.make_async_copy(table.at[srcv.at[1]], rows1, sem1).start()

        def body(jj, carry):
            j = 2 * jj
            pltpu.make_async_copy(table.at[srcv.at[j]], rows0, sem0).wait()
            pltpu.sync_copy(rows0, acc.at[dstv.at[j]], add=True)
            pltpu.make_async_copy(table.at[srcv.at[j + 2]], rows0, sem0).start()
            pltpu.make_async_copy(table.at[srcv.at[j + 1]], rows1, sem1).wait()
            pltpu.sync_copy(rows1, acc.at[dstv.at[j + 1]], add=True)
            pltpu.make_async_copy(table.at[srcv.at[j + 3]], rows1, sem1).start()
            return carry

        lax.fori_loop(0, KCH // 2, body, 0)
        pltpu.make_async_copy(table.at[srcv.at[KCH]], rows0, sem0).wait()
        pltpu.make_async_copy(table.at[srcv.at[KCH + 1]], rows1, sem1).wait()
        plsc.subcore_barrier()
        pltpu.sync_copy(acc.at[sl], out.at[c, sl])

    return k


def _tc_matmul_hs(x, w_all, degp):
    def body(x_ref, w_ref, d_ref, hs_ref, dis_ref):
        deg = jnp.sum(d_ref[...], axis=0)[:, None] + 1.0
        dis = lax.rsqrt(deg)
        h = jnp.dot(x_ref[...], w_ref[...], preferred_element_type=jnp.float32)
        hs_ref[...] = h * dis
        dis_ref[...] = dis

    return pl.pallas_call(
        body,
        grid=(NBLK,),
        in_specs=[
            pl.BlockSpec((ROWS_PER_TILE, IN_CH), lambda i: (i, 0)),
            pl.BlockSpec((IN_CH, CH2), lambda i: (0, 0)),
            pl.BlockSpec((NW, ROWS_PER_TILE), lambda i: (0, i)),
        ],
        out_specs=[
            pl.BlockSpec((ROWS_PER_TILE, CH2), lambda i: (i, 0)),
            pl.BlockSpec((ROWS_PER_TILE, 1), lambda i: (i, 0)),
        ],
        out_shape=[
            jax.ShapeDtypeStruct((R, CH2), jnp.float32),
            jax.ShapeDtypeStruct((R, 1), jnp.float32),
        ],
    )(x, w_all, degp)


FBLK = 10
FROWS = N // FBLK


def _tc_finalize(agg, hs, dis, b_all):
    def body(a_ref, hs_ref, dis_ref, b_ref, mu_ref, ls_ref):
        o = dis_ref[...] * (a_ref[0] + a_ref[1] + hs_ref[...]) + b_ref[...]
        mu_ref[...] = o[:, :16]
        ls_ref[...] = o[:, 16:]

    return pl.pallas_call(
        body,
        grid=(FBLK,),
        in_specs=[
            pl.BlockSpec((2, FROWS, CH2), lambda i: (0, i, 0)),
            pl.BlockSpec((FROWS, CH2), lambda i: (i, 0)),
            pl.BlockSpec((FROWS, 1), lambda i: (i, 0)),
            pl.BlockSpec((1, CH2), lambda i: (0, 0)),
        ],
        out_specs=[
            pl.BlockSpec((FROWS, 16), lambda i: (i, 0)),
            pl.BlockSpec((FROWS, 16), lambda i: (i, 0)),
        ],
        out_shape=[
            jax.ShapeDtypeStruct((N, 16), jnp.float32),
            jax.ShapeDtypeStruct((N, 16), jnp.float32),
        ],
    )(agg, hs, dis, b_all)


def kernel(x, edge_index, W_mu, b_mu, W_logstd, b_logstd):
    npad = NW * KCH * CHUNK - E
    pad_ids = (N + jnp.arange(npad, dtype=jnp.int32) % (R - N)).astype(jnp.int32)
    pad2 = jnp.broadcast_to(pad_ids[None], (2, npad))
    dummy = jnp.broadcast_to(
        (N + jnp.arange(NW * 2 * CHUNK, dtype=jnp.int32) % (R - N)).reshape(
            1, NW, 2, CHUNK), (2, NW, 2, CHUNK))
    edges = jnp.concatenate(
        [jnp.concatenate([edge_index, pad2], axis=1).reshape(2, NW, KCH, CHUNK),
         dummy], axis=2)

    zer1 = jnp.zeros((R,), jnp.float32)
    degp = _sc_deg()(edges, zer1)

    w_all = jnp.concatenate([W_mu, W_logstd], axis=0).T
    hs, dis = _tc_matmul_hs(x, w_all, degp)

    zer32 = jnp.zeros((R, CH2), jnp.float32)
    agg = _sc_edge_agg()(hs, edges, zer32)

    b_all = jnp.concatenate([b_mu, b_logstd]).reshape(1, CH2)
    return tuple(_tc_finalize(agg, hs, dis, b_all))


# speedup vs baseline: 79.0472x; 1.1277x over previous
import functools

import jax
import jax.numpy as jnp
from jax import lax
from jax.experimental import pallas as pl
from jax.experimental.pallas import tpu as pltpu
from jax.experimental.pallas import tpu_sc as plsc

N = 10000
E = 320000
IN_CH = 128
CH2 = 32
R = 10240
NW = 32
CHUNK = 128
KCH = 80
KIDX = KCH + 2
E_PAD = NW * KIDX * CHUNK
ROWS_PER_TILE = R // 16
NBLK = 16

_SC_PARAMS = pltpu.CompilerParams(
    use_tc_tiling_on_sc=False, needs_layout_passes=False)


@functools.cache
def _sc_deg():
    mesh = plsc.VectorSubcoreMesh(core_axis_name="c", subcore_axis_name="s")

    @functools.partial(
        pl.kernel,
        mesh=mesh,
        compiler_params=_SC_PARAMS,
        out_type=jax.ShapeDtypeStruct((NW, R), jnp.float32),
        scratch_types=[
            pltpu.VMEM((KIDX, CHUNK), jnp.int32),
            pltpu.VMEM((R,), jnp.float32),
        ],
    )
    def k(edges, zer, out, dstv, acc):
        c = lax.axis_index("c")
        s = lax.axis_index("s")
        wid = s * 2 + c
        pltpu.sync_copy(zer, acc)
        pltpu.sync_copy(edges.at[1, wid], dstv)
        ones16 = jnp.ones((16,), jnp.float32)

        def body(j, carry):
            def inner(i, carry2):
                idx16 = dstv[j, pl.ds(i * 16, 16)]
                plsc.addupdate_scatter(acc, [idx16], ones16)
                return carry2

            return lax.fori_loop(0, CHUNK // 16, inner, carry)

        lax.fori_loop(0, KCH, body, 0)
        pltpu.sync_copy(acc, out.at[wid])

    return k


@functools.cache
def _sc_edge_agg():
    mesh = plsc.VectorSubcoreMesh(core_axis_name="c", subcore_axis_name="s")

    @functools.partial(
        pl.kernel,
        mesh=mesh,
        compiler_params=_SC_PARAMS,
        out_type=jax.ShapeDtypeStruct((2, R, CH2), jnp.float32),
        scratch_types=[
            pltpu.VMEM((KIDX, CHUNK), jnp.int32),
            pltpu.VMEM((KIDX, CHUNK), jnp.int32),
            [pltpu.VMEM((CHUNK, CH2), jnp.float32)] * 4,
            pltpu.VMEM_SHARED((R, CH2), jnp.float32),
            [pltpu.SemaphoreType.DMA] * 4,
            [pltpu.SemaphoreType.DMA] * 4,
        ],
    )
    def k(table, edges, zer, out, srcv, dstv, rows, acc, gsem, ssem):
        c = lax.axis_index("c")
        s = lax.axis_index("s")
        wid = s * 2 + c
        sl = pl.ds(s * ROWS_PER_TILE, ROWS_PER_TILE)
        pltpu.sync_copy(zer.at[sl], acc.at[sl])
        pltpu.sync_copy(edges.at[0, wid], srcv)
        pltpu.sync_copy(edges.at[1, wid], dstv)
        plsc.subcore_barrier()

        def gstart(j, b):
            pltpu.async_copy(table.at[srcv.at[j]], rows[b], gsem[b])

        def gwait(j, b):
            pltpu.make_async_copy(table.at[srcv.at[j]], rows[b], gsem[b]).wait()

        def sstart(j, b):
            pltpu.async_copy(rows[b], acc.at[dstv.at[j]], ssem[b], add=True)

        def swait(j, b):
            pltpu.make_async_copy(rows[b], acc.at[dstv.at[j]], ssem[b]).wait()

        def slot(j, b):
            swait(j - 2, (b + 2) % 4)
            gstart(j + 2, (b + 2) % 4)
            gwait(j, b)
            sstart(j, b)

        gstart(0, 0)
        gstart(1, 1)
        gstart(2, 2)
        gwait(0, 0)
        sstart(0, 0)
        gstart(3, 3)
        gwait(1, 1)
        sstart(1, 1)

        def body(jj, carry):
            j = 4 * jj + 2
            slot(j, 2)
            slot(j + 1, 3)
            slot(j + 2, 0)
            slot(j + 3, 1)
            return carry

        lax.fori_loop(0, (KCH - 4) // 4, body, 0)
        slot(KCH - 2, 2)
        slot(KCH - 1, 3)
        swait(KCH - 2, 2)
        swait(KCH - 1, 3)
        gwait(KCH, 0)
        gwait(KCH + 1, 1)
        plsc.subcore_barrier()
        pltpu.sync_copy(acc.at[sl], out.at[c, sl])

    return k


def _tc_matmul_hs(x, w_all, degp):
    def body(x_ref, w_ref, d_ref, hs_ref, dis_ref):
        deg = jnp.sum(d_ref[...], axis=0)[:, None] + 1.0
        dis = lax.rsqrt(deg)
        h = jnp.dot(x_ref[...], w_ref[...], preferred_element_type=jnp.float32)
        hs_ref[...] = h * dis
        dis_ref[...] = dis

    return pl.pallas_call(
        body,
        grid=(NBLK,),
        in_specs=[
            pl.BlockSpec((ROWS_PER_TILE, IN_CH), lambda i: (i, 0)),
            pl.BlockSpec((IN_CH, CH2), lambda i: (0, 0)),
            pl.BlockSpec((NW, ROWS_PER_TILE), lambda i: (0, i)),
        ],
        out_specs=[
            pl.BlockSpec((ROWS_PER_TILE, CH2), lambda i: (i, 0)),
            pl.BlockSpec((ROWS_PER_TILE, 1), lambda i: (i, 0)),
        ],
        out_shape=[
            jax.ShapeDtypeStruct((R, CH2), jnp.float32),
            jax.ShapeDtypeStruct((R, 1), jnp.float32),
        ],
    )(x, w_all, degp)


FBLK = 10
FROWS = N // FBLK


def _tc_finalize(agg, hs, dis, b_all):
    def body(a_ref, hs_ref, dis_ref, b_ref, mu_ref, ls_ref):
        o = dis_ref[...] * (a_ref[0] + a_ref[1] + hs_ref[...]) + b_ref[...]
        mu_ref[...] = o[:, :16]
        ls_ref[...] = o[:, 16:]

    return pl.pallas_call(
        body,
        grid=(FBLK,),
        in_specs=[
            pl.BlockSpec((2, FROWS, CH2), lambda i: (0, i, 0)),
            pl.BlockSpec((FROWS, CH2), lambda i: (i, 0)),
            pl.BlockSpec((FROWS, 1), lambda i: (i, 0)),
            pl.BlockSpec((1, CH2), lambda i: (0, 0)),
        ],
        out_specs=[
            pl.BlockSpec((FROWS, 16), lambda i: (i, 0)),
            pl.BlockSpec((FROWS, 16), lambda i: (i, 0)),
        ],
        out_shape=[
            jax.ShapeDtypeStruct((N, 16), jnp.float32),
            jax.ShapeDtypeStruct((N, 16), jnp.float32),
        ],
    )(agg, hs, dis, b_all)


def kernel(x, edge_index, W_mu, b_mu, W_logstd, b_logstd):
    npad = NW * KCH * CHUNK - E
    pad_ids = (N + jnp.arange(npad, dtype=jnp.int32) % (R - N)).astype(jnp.int32)
    pad2 = jnp.broadcast_to(pad_ids[None], (2, npad))
    dummy = jnp.broadcast_to(
        (N + jnp.arange(NW * 2 * CHUNK, dtype=jnp.int32) % (R - N)).reshape(
            1, NW, 2, CHUNK), (2, NW, 2, CHUNK))
    edges = jnp.concatenate(
        [jnp.concatenate([edge_index, pad2], axis=1).reshape(2, NW, KCH, CHUNK),
         dummy], axis=2)

    zer1 = jnp.zeros((R,), jnp.float32)
    degp = _sc_deg()(edges, zer1)

    w_all = jnp.concatenate([W_mu, W_logstd], axis=0).T
    hs, dis = _tc_matmul_hs(x, w_all, degp)

    zer32 = jnp.zeros((R, CH2), jnp.float32)
    agg = _sc_edge_agg()(hs, edges, zer32)

    b_all = jnp.concatenate([b_mu, b_logstd]).reshape(1, CH2)
    return tuple(_tc_finalize(agg, hs, dis, b_all))


# speedup vs baseline: 85.6632x; 1.0837x over previous
import functools

import jax
import jax.numpy as jnp
from jax import lax
from jax.experimental import pallas as pl
from jax.experimental.pallas import tpu as pltpu
from jax.experimental.pallas import tpu_sc as plsc

N = 10000
E = 320000
IN_CH = 128
CH2 = 32
R = 10240
NW = 32
CHUNK = 128
KCH = 80
ROWS_PER_TILE = R // 16

_SC_PARAMS = pltpu.CompilerParams(
    use_tc_tiling_on_sc=False, needs_layout_passes=False)


@functools.cache
def _sc_deg():
    mesh = plsc.VectorSubcoreMesh(core_axis_name="c", subcore_axis_name="s")

    @functools.partial(
        pl.kernel,
        mesh=mesh,
        compiler_params=_SC_PARAMS,
        out_type=jax.ShapeDtypeStruct((NW, R), jnp.float32),
        scratch_types=[
            pltpu.VMEM((KCH, CHUNK), jnp.int32),
            pltpu.VMEM((R,), jnp.float32),
        ],
    )
    def k(edges, zer, out, dstv, acc):
        c = lax.axis_index("c")
        s = lax.axis_index("s")
        wid = s * 2 + c
        pltpu.sync_copy(zer, acc)
        pltpu.sync_copy(edges.at[1, wid], dstv)
        ones16 = jnp.ones((16,), jnp.float32)

        def body(j, carry):
            def inner(i, carry2):
                idx16 = dstv[j, pl.ds(i * 16, 16)]
                plsc.addupdate_scatter(acc, [idx16], ones16)
                return carry2

            return lax.fori_loop(0, CHUNK // 16, inner, carry)

        lax.fori_loop(0, KCH, body, 0)
        pltpu.sync_copy(acc, out.at[wid])

    return k


@functools.cache
def _sc_edge_agg():
    mesh = plsc.VectorSubcoreMesh(core_axis_name="c", subcore_axis_name="s")

    @functools.partial(
        pl.kernel,
        mesh=mesh,
        compiler_params=_SC_PARAMS,
        out_type=jax.ShapeDtypeStruct((2, R, CH2), jnp.float32),
        scratch_types=[
            pltpu.VMEM((KCH, CHUNK), jnp.int32),
            pltpu.VMEM((KCH, CHUNK), jnp.int32),
            [pltpu.VMEM((CHUNK, CH2), jnp.float32)] * 4,
            pltpu.VMEM_SHARED((R, CH2), jnp.float32),
            [pltpu.SemaphoreType.DMA] * 4,
            [pltpu.SemaphoreType.DMA] * 4,
        ],
    )
    def k(table, edges, zer, out, srcv, dstv, rows, acc, gsem, ssem):
        c = lax.axis_index("c")
        s = lax.axis_index("s")
        wid = s * 2 + c
        sl = pl.ds(s * ROWS_PER_TILE, ROWS_PER_TILE)
        pltpu.sync_copy(zer.at[sl], acc.at[sl])
        pltpu.sync_copy(edges.at[0, wid], srcv)
        pltpu.sync_copy(edges.at[1, wid], dstv)
        plsc.subcore_barrier()

        def gstart(j, b):
            pltpu.async_copy(table.at[srcv.at[j]], rows[b], gsem[b])

        def gwait(j, b):
            pltpu.make_async_copy(table.at[srcv.at[j]], rows[b], gsem[b]).wait()

        def sstart(j, b):
            pltpu.async_copy(rows[b], acc.at[dstv.at[j]], ssem[b], add=True)

        def swait(j, b):
            pltpu.make_async_copy(rows[b], acc.at[dstv.at[j]], ssem[b]).wait()

        def slot(j, b):
            swait(j - 2, (b + 2) % 4)
            gstart(j + 2, (b + 2) % 4)
            gwait(j, b)
            sstart(j, b)

        gstart(0, 0)
        gstart(1, 1)
        gstart(2, 2)
        gwait(0, 0)
        sstart(0, 0)
        gstart(3, 3)
        gwait(1, 1)
        sstart(1, 1)

        def body(jj, carry):
            j = 4 * jj + 2
            slot(j, 2)
            slot(j + 1, 3)
            slot(j + 2, 0)
            slot(j + 3, 1)
            return carry

        lax.fori_loop(0, (KCH - 4) // 4, body, 0)
        swait(KCH - 4, 0)
        gwait(KCH - 2, 2)
        sstart(KCH - 2, 2)
        swait(KCH - 3, 1)
        gwait(KCH - 1, 3)
        sstart(KCH - 1, 3)
        swait(KCH - 2, 2)
        swait(KCH - 1, 3)
        plsc.subcore_barrier()
        pltpu.sync_copy(acc.at[sl], out.at[c, sl])

    return k


def _tc_matmul_hs(x, w_all, degp):
    def body(x_ref, w_ref, d_ref, hs_ref, dis_ref):
        deg = jnp.sum(d_ref[...], axis=0)[:N, None] + 1.0
        dis = lax.rsqrt(deg)
        h = jnp.dot(x_ref[...], w_ref[...], preferred_element_type=jnp.float32)
        hs_ref[...] = h * dis
        dis_ref[...] = dis

    return pl.pallas_call(
        body,
        out_shape=[
            jax.ShapeDtypeStruct((N, CH2), jnp.float32),
            jax.ShapeDtypeStruct((N, 1), jnp.float32),
        ],
    )(x, w_all, degp)


def _tc_finalize(agg, hs, dis, b_all):
    def body(a_ref, hs_ref, dis_ref, b_ref, mu_ref, ls_ref):
        s = a_ref[0, :N, :] + a_ref[1, :N, :] + hs_ref[...]
        o = dis_ref[...] * s + b_ref[...]
        mu_ref[...] = o[:, :16]
        ls_ref[...] = o[:, 16:]

    return pl.pallas_call(
        body,
        out_shape=[
            jax.ShapeDtypeStruct((N, 16), jnp.float32),
            jax.ShapeDtypeStruct((N, 16), jnp.float32),
        ],
    )(agg, hs, dis, b_all)


def kernel(x, edge_index, W_mu, b_mu, W_logstd, b_logstd):
    npad = NW * KCH * CHUNK - E
    pad_src = jnp.arange(npad, dtype=jnp.int32) % N
    pad_dst = N + jnp.arange(npad, dtype=jnp.int32) % (R - N)
    pad2 = jnp.stack([pad_src, pad_dst])
    edges = jnp.concatenate([edge_index, pad2], axis=1).reshape(
        2, NW, KCH, CHUNK)

    zer1 = jnp.zeros((R,), jnp.float32)
    degp = _sc_deg()(edges, zer1)

    w_all = jnp.concatenate([W_mu, W_logstd], axis=0).T
    hs, dis = _tc_matmul_hs(x, w_all, degp)

    zer32 = jnp.zeros((R, CH2), jnp.float32)
    agg = _sc_edge_agg()(hs, edges, zer32)

    b_all = jnp.concatenate([b_mu, b_logstd]).reshape(1, CH2)
    return tuple(_tc_finalize(agg, hs, dis, b_all))
